# baseline (speedup 1.0000x reference)
import functools

import jax
import jax.numpy as jnp
from jax import lax
from jax.experimental import pallas as pl
from jax.experimental.pallas import tpu as pltpu
from jax.experimental.pallas import tpu_sc as plsc

N_NODES = 100000
FEAT = 128
N_GRAPHS = 512

NC = 2
NS = 16
NW = NC * NS

CHUNK = 400
NSUB = 4
SUB = CHUNK // NSUB
N_CHUNKS = N_NODES // CHUNK
T_FULL = N_CHUNKS // NW
N_TAIL = N_CHUNKS - T_FULL * NW
ROWS_PER_TILE = N_GRAPHS // NS
NBUF = 2


def _sc_body(x_hbm, b_hbm, out_hbm, xbuf, zbuf, idxbuf, acc, semx, semi, sems):
    c = lax.axis_index("c")
    s = lax.axis_index("s")
    wid = c * NS + s

    def fire_copy(t, b):
        j = wid + t * NW
        dx = pltpu.async_copy(x_hbm.at[pl.ds(j * CHUNK, CHUNK)], xbuf.at[b],
                              semx[b])
        di = pltpu.async_copy(b_hbm.at[j], idxbuf.at[b], semi[b])
        return dx, di

    def fire_scatter(b):
        return [
            pltpu.async_copy(xbuf.at[b, pl.ds(u * SUB, SUB)],
                             acc.at[idxbuf.at[b, u]], sems[b], add=True)
            for u in range(NSUB)
        ]

    cdescs = [None] * NBUF
    sdescs = [None] * NBUF
    for t in range(min(2, T_FULL)):
        cdescs[t % NBUF] = fire_copy(t, t % NBUF)

    def zero_row(i, carry):
        for l in range(FEAT // 16):
            zbuf[i, pl.ds(l * 16, 16)] = jnp.zeros((16,), jnp.float32)
        return carry

    lax.fori_loop(0, ROWS_PER_TILE, zero_row, 0)
    pltpu.sync_copy(zbuf, acc.at[pl.ds(s * ROWS_PER_TILE, ROWS_PER_TILE)])
    plsc.subcore_barrier()

    for t in range(T_FULL):
        b = t % NBUF
        dx, di = cdescs[b]
        dx.wait()
        di.wait()
        sdescs[b] = fire_scatter(b)
        tn = t + 2
        if tn < T_FULL:
            bn = tn % NBUF
            if sdescs[bn] is not None:
                for d in sdescs[bn]:
                    d.wait()
                sdescs[bn] = None
            cdescs[bn] = fire_copy(tn, bn)
    for b in range(NBUF):
        if sdescs[b] is not None:
            for d in sdescs[b]:
                d.wait()

    @pl.when(wid < N_TAIL)
    def _():
        j = wid + T_FULL * NW
        pltpu.sync_copy(x_hbm.at[pl.ds(j * CHUNK, CHUNK)], xbuf.at[0])
        pltpu.sync_copy(b_hbm.at[j], idxbuf.at[0])
        for u in range(NSUB):
            pltpu.sync_copy(xbuf.at[0, pl.ds(u * SUB, SUB)],
                            acc.at[idxbuf.at[0, u]], add=True)

    plsc.subcore_barrier()

    pltpu.sync_copy(acc.at[pl.ds(s * ROWS_PER_TILE, ROWS_PER_TILE)], zbuf)
    pltpu.sync_copy(zbuf,
                    out_hbm.at[c, pl.ds(s * ROWS_PER_TILE, ROWS_PER_TILE)])


@functools.partial(
    pl.kernel,
    out_type=jax.ShapeDtypeStruct((NC, N_GRAPHS, FEAT), jnp.float32),
    mesh=plsc.VectorSubcoreMesh(core_axis_name="c", subcore_axis_name="s"),
    scratch_types=[
        pltpu.VMEM((NBUF, CHUNK, FEAT), jnp.float32),
        pltpu.VMEM((ROWS_PER_TILE, FEAT), jnp.float32),
        pltpu.VMEM((NBUF, NSUB, SUB), jnp.int32),
        pltpu.VMEM_SHARED((N_GRAPHS, FEAT), jnp.float32),
        [pltpu.SemaphoreType.DMA] * NBUF,
        [pltpu.SemaphoreType.DMA] * NBUF,
        [pltpu.SemaphoreType.DMA] * NBUF,
    ],
)
def _sc_segment_sum(x_hbm, b_hbm, out_hbm, xbuf, zbuf, idxbuf, acc,
                    semx, semi, sems):
    _sc_body(x_hbm, b_hbm, out_hbm, xbuf, zbuf, idxbuf, acc, semx, semi, sems)


def _add_body(p_ref, o_ref):
    o_ref[...] = p_ref[0] + p_ref[1]


_merge = pl.pallas_call(
    _add_body,
    out_shape=jax.ShapeDtypeStruct((N_GRAPHS, FEAT), jnp.float32),
)


@jax.jit
def kernel(x, batch):
    batch3 = batch.reshape(N_CHUNKS, NSUB, SUB)
    partials = _sc_segment_sum(x, batch3)
    return _merge(partials)


# speedup vs baseline: 1.0025x; 1.0025x over previous
import functools

import jax
import jax.numpy as jnp
from jax import lax
from jax.experimental import pallas as pl
from jax.experimental.pallas import tpu as pltpu
from jax.experimental.pallas import tpu_sc as plsc

N_NODES = 100000
FEAT = 128
N_GRAPHS = 512

NC = 2
NS = 16
NW = NC * NS

CHUNK = 160
NSUB = 2
SUB = CHUNK // NSUB
N_CHUNKS = N_NODES // CHUNK
T_FULL = N_CHUNKS // NW
N_TAIL = N_CHUNKS - T_FULL * NW
ROWS_PER_TILE = N_GRAPHS // NS
NBUF = 5
LEAD = 3


def _sc_body(x_hbm, b_hbm, out_hbm, xbuf, zbuf, idxbuf, acc, semx, sems):
    c = lax.axis_index("c")
    s = lax.axis_index("s")
    wid = c * NS + s
    start = wid * T_FULL + jnp.minimum(wid, N_TAIL)

    def fire_copy(t, b):
        return pltpu.async_copy(
            x_hbm.at[pl.ds((start + t) * CHUNK, CHUNK)], xbuf.at[b], semx[b])

    def fire_scatter(t, b):
        return [
            pltpu.async_copy(xbuf.at[b, pl.ds(u * SUB, SUB)],
                             acc.at[idxbuf.at[t, u]], sems[b], add=True)
            for u in range(NSUB)
        ]

    cdescs = [None] * NBUF
    sdescs = [None] * NBUF
    for t in range(LEAD):
        cdescs[t % NBUF] = fire_copy(t, t % NBUF)

    pltpu.sync_copy(b_hbm.at[pl.ds(start, T_FULL)],
                    idxbuf.at[pl.ds(0, T_FULL)])

    def zero_row(i, carry):
        for l in range(FEAT // 16):
            zbuf[i, pl.ds(l * 16, 16)] = jnp.zeros((16,), jnp.float32)
        return carry

    lax.fori_loop(0, ROWS_PER_TILE, zero_row, 0)
    pltpu.sync_copy(zbuf, acc.at[pl.ds(s * ROWS_PER_TILE, ROWS_PER_TILE)])
    plsc.subcore_barrier()

    for t in range(T_FULL):
        b = t % NBUF
        cdescs[b].wait()
        sdescs[b] = fire_scatter(t, b)
        tn = t + LEAD
        if tn < T_FULL:
            bn = tn % NBUF
            if sdescs[bn] is not None:
                for d in sdescs[bn]:
                    d.wait()
                sdescs[bn] = None
            cdescs[bn] = fire_copy(tn, bn)
    for b in range(NBUF):
        if sdescs[b] is not None:
            for d in sdescs[b]:
                d.wait()

    @pl.when(wid < N_TAIL)
    def _():
        pltpu.sync_copy(b_hbm.at[start + T_FULL], idxbuf.at[T_FULL])
        pltpu.sync_copy(x_hbm.at[pl.ds((start + T_FULL) * CHUNK, CHUNK)],
                        xbuf.at[0])
        for u in range(NSUB):
            pltpu.sync_copy(xbuf.at[0, pl.ds(u * SUB, SUB)],
                            acc.at[idxbuf.at[T_FULL, u]], add=True)

    plsc.subcore_barrier()

    pltpu.sync_copy(acc.at[pl.ds(s * ROWS_PER_TILE, ROWS_PER_TILE)], zbuf)
    pltpu.sync_copy(zbuf,
                    out_hbm.at[c, pl.ds(s * ROWS_PER_TILE, ROWS_PER_TILE)])


@functools.partial(
    pl.kernel,
    out_type=jax.ShapeDtypeStruct((NC, N_GRAPHS, FEAT), jnp.float32),
    mesh=plsc.VectorSubcoreMesh(core_axis_name="c", subcore_axis_name="s"),
    scratch_types=[
        pltpu.VMEM((NBUF, CHUNK, FEAT), jnp.float32),
        pltpu.VMEM((ROWS_PER_TILE, FEAT), jnp.float32),
        pltpu.VMEM((T_FULL + 1, NSUB, SUB), jnp.int32),
        pltpu.VMEM_SHARED((N_GRAPHS, FEAT), jnp.float32),
        [pltpu.SemaphoreType.DMA] * NBUF,
        [pltpu.SemaphoreType.DMA] * NBUF,
    ],
)
def _sc_segment_sum(x_hbm, b_hbm, out_hbm, xbuf, zbuf, idxbuf, acc,
                    semx, sems):
    _sc_body(x_hbm, b_hbm, out_hbm, xbuf, zbuf, idxbuf, acc, semx, sems)


def _add_body(p_ref, o_ref):
    o_ref[...] = p_ref[0] + p_ref[1]


_merge = pl.pallas_call(
    _add_body,
    out_shape=jax.ShapeDtypeStruct((N_GRAPHS, FEAT), jnp.float32),
)


@jax.jit
def kernel(x, batch):
    batch3 = batch.reshape(N_CHUNKS, NSUB, SUB)
    partials = _sc_segment_sum(x, batch3)
    return _merge(partials)


# speedup vs baseline: 1.0116x; 1.0091x over previous
import functools

import jax
import jax.numpy as jnp
from jax import lax
from jax.experimental import pallas as pl
from jax.experimental.pallas import tpu as pltpu
from jax.experimental.pallas import tpu_sc as plsc

N_NODES = 100000
FEAT = 128
N_GRAPHS = 512

NC = 2
NS = 16
NW = NC * NS

CHUNK = 200
NSUB = 2
SUB = CHUNK // NSUB
N_CHUNKS = N_NODES // CHUNK
T_FULL = N_CHUNKS // NW
N_TAIL = N_CHUNKS - T_FULL * NW
ROWS_PER_TILE = N_GRAPHS // NS
NBUF = 4
LEAD = 2


def _sc_body(x_hbm, b_hbm, out_hbm, xbuf, zbuf, idxbuf, acc, semx, sems):
    c = lax.axis_index("c")
    s = lax.axis_index("s")
    wid = c * NS + s
    start = wid * T_FULL + jnp.minimum(wid, N_TAIL)

    def fire_copy(t, b):
        return pltpu.async_copy(
            x_hbm.at[pl.ds((start + t) * CHUNK, CHUNK)], xbuf.at[b], semx[b])

    def fire_scatter(t, b):
        return [
            pltpu.async_copy(xbuf.at[b, pl.ds(u * SUB, SUB)],
                             acc.at[idxbuf.at[t, u]], sems[b], add=True)
            for u in range(NSUB)
        ]

    cdescs = [None] * NBUF
    sdescs = [None] * NBUF
    for t in range(LEAD):
        cdescs[t % NBUF] = fire_copy(t, t % NBUF)

    pltpu.sync_copy(b_hbm.at[pl.ds(start, T_FULL)],
                    idxbuf.at[pl.ds(0, T_FULL)])

    def zero_row(i, carry):
        for l in range(FEAT // 16):
            zbuf[i, pl.ds(l * 16, 16)] = jnp.zeros((16,), jnp.float32)
        return carry

    lax.fori_loop(0, ROWS_PER_TILE, zero_row, 0)
    pltpu.sync_copy(zbuf, acc.at[pl.ds(s * ROWS_PER_TILE, ROWS_PER_TILE)])
    plsc.subcore_barrier()

    for t in range(T_FULL):
        b = t % NBUF
        cdescs[b].wait()
        sdescs[b] = fire_scatter(t, b)
        tn = t + LEAD
        if tn < T_FULL:
            bn = tn % NBUF
            if sdescs[bn] is not None:
                for d in sdescs[bn]:
                    d.wait()
                sdescs[bn] = None
            cdescs[bn] = fire_copy(tn, bn)
    for b in range(NBUF):
        if sdescs[b] is not None:
            for d in sdescs[b]:
                d.wait()

    @pl.when(wid < N_TAIL)
    def _():
        pltpu.sync_copy(b_hbm.at[start + T_FULL], idxbuf.at[T_FULL])
        pltpu.sync_copy(x_hbm.at[pl.ds((start + T_FULL) * CHUNK, CHUNK)],
                        xbuf.at[0])
        for u in range(NSUB):
            pltpu.sync_copy(xbuf.at[0, pl.ds(u * SUB, SUB)],
                            acc.at[idxbuf.at[T_FULL, u]], add=True)

    plsc.subcore_barrier()

    pltpu.sync_copy(acc.at[pl.ds(s * ROWS_PER_TILE, ROWS_PER_TILE)], zbuf)
    pltpu.sync_copy(zbuf,
                    out_hbm.at[c, pl.ds(s * ROWS_PER_TILE, ROWS_PER_TILE)])


@functools.partial(
    pl.kernel,
    out_type=jax.ShapeDtypeStruct((NC, N_GRAPHS, FEAT), jnp.float32),
    mesh=plsc.VectorSubcoreMesh(core_axis_name="c", subcore_axis_name="s"),
    scratch_types=[
        pltpu.VMEM((NBUF, CHUNK, FEAT), jnp.float32),
        pltpu.VMEM((ROWS_PER_TILE, FEAT), jnp.float32),
        pltpu.VMEM((T_FULL + 1, NSUB, SUB), jnp.int32),
        pltpu.VMEM_SHARED((N_GRAPHS, FEAT), jnp.float32),
        [pltpu.SemaphoreType.DMA] * NBUF,
        [pltpu.SemaphoreType.DMA] * NBUF,
    ],
)
def _sc_segment_sum(x_hbm, b_hbm, out_hbm, xbuf, zbuf, idxbuf, acc,
                    semx, sems):
    _sc_body(x_hbm, b_hbm, out_hbm, xbuf, zbuf, idxbuf, acc, semx, sems)


def _add_body(p_ref, o_ref):
    o_ref[...] = p_ref[0] + p_ref[1]


_merge = pl.pallas_call(
    _add_body,
    out_shape=jax.ShapeDtypeStruct((N_GRAPHS, FEAT), jnp.float32),
)


@jax.jit
def kernel(x, batch):
    batch3 = batch.reshape(N_CHUNKS, NSUB, SUB)
    partials = _sc_segment_sum(x, batch3)
    return _merge(partials)


# speedup vs baseline: 1.0416x; 1.0296x over previous
import functools

import jax
import jax.numpy as jnp
from jax import lax
from jax.experimental import pallas as pl
from jax.experimental.pallas import tpu as pltpu
from jax.experimental.pallas import tpu_sc as plsc

N_NODES = 100000
FEAT = 128
N_GRAPHS = 512

NC = 2
NS = 16
NW = NC * NS

CHUNK = 200
NSUB = 2
SUB = CHUNK // NSUB
N_CHUNKS = N_NODES // CHUNK
T_FULL = N_CHUNKS // NW
N_TAIL = N_CHUNKS - T_FULL * NW
ROWS_PER_TILE = N_GRAPHS // NS
NBUF = 4


def _sc_body(x_hbm, b_hbm, out_hbm, xbuf, zbuf, idxbuf, acc, semx, semi, sems):
    c = lax.axis_index("c")
    s = lax.axis_index("s")
    wid = c * NS + s

    def fire_copy(t, b):
        j = wid + t * NW
        dx = pltpu.async_copy(x_hbm.at[pl.ds(j * CHUNK, CHUNK)], xbuf.at[b],
                              semx[b])
        di = pltpu.async_copy(b_hbm.at[j], idxbuf.at[b], semi[b])
        return dx, di

    def fire_scatter(b):
        return [
            pltpu.async_copy(xbuf.at[b, pl.ds(u * SUB, SUB)],
                             acc.at[idxbuf.at[b, u]], sems[b], add=True)
            for u in range(NSUB)
        ]

    cdescs = [None] * NBUF
    sdescs = [None] * NBUF
    for t in range(min(2, T_FULL)):
        cdescs[t % NBUF] = fire_copy(t, t % NBUF)

    def zero_row(i, carry):
        for l in range(FEAT // 16):
            zbuf[i, pl.ds(l * 16, 16)] = jnp.zeros((16,), jnp.float32)
        return carry

    lax.fori_loop(0, ROWS_PER_TILE, zero_row, 0)
    pltpu.sync_copy(zbuf, acc.at[pl.ds(s * ROWS_PER_TILE, ROWS_PER_TILE)])
    plsc.subcore_barrier()

    for t in range(T_FULL):
        b = t % NBUF
        dx, di = cdescs[b]
        dx.wait()
        di.wait()
        sdescs[b] = fire_scatter(b)
        tn = t + 2
        if tn < T_FULL:
            bn = tn % NBUF
            if sdescs[bn] is not None:
                for d in sdescs[bn]:
                    d.wait()
                sdescs[bn] = None
            cdescs[bn] = fire_copy(tn, bn)
    for b in range(NBUF):
        if sdescs[b] is not None:
            for d in sdescs[b]:
                d.wait()

    @pl.when(wid < N_TAIL)
    def _():
        j = wid + T_FULL * NW
        pltpu.sync_copy(x_hbm.at[pl.ds(j * CHUNK, CHUNK)], xbuf.at[0])
        pltpu.sync_copy(b_hbm.at[j], idxbuf.at[0])
        for u in range(NSUB):
            pltpu.sync_copy(xbuf.at[0, pl.ds(u * SUB, SUB)],
                            acc.at[idxbuf.at[0, u]], add=True)

    plsc.subcore_barrier()

    pltpu.sync_copy(acc.at[pl.ds(s * ROWS_PER_TILE, ROWS_PER_TILE)],
                    out_hbm.at[c, pl.ds(s * ROWS_PER_TILE, ROWS_PER_TILE)])


@functools.partial(
    pl.kernel,
    out_type=jax.ShapeDtypeStruct((NC, N_GRAPHS, FEAT), jnp.float32),
    mesh=plsc.VectorSubcoreMesh(core_axis_name="c", subcore_axis_name="s"),
    scratch_types=[
        pltpu.VMEM((NBUF, CHUNK, FEAT), jnp.float32),
        pltpu.VMEM((ROWS_PER_TILE, FEAT), jnp.float32),
        pltpu.VMEM((NBUF, NSUB, SUB), jnp.int32),
        pltpu.VMEM_SHARED((N_GRAPHS, FEAT), jnp.float32),
        [pltpu.SemaphoreType.DMA] * NBUF,
        [pltpu.SemaphoreType.DMA] * NBUF,
        [pltpu.SemaphoreType.DMA] * NBUF,
    ],
)
def _sc_segment_sum(x_hbm, b_hbm, out_hbm, xbuf, zbuf, idxbuf, acc,
                    semx, semi, sems):
    _sc_body(x_hbm, b_hbm, out_hbm, xbuf, zbuf, idxbuf, acc, semx, semi, sems)


def _add_body(p_ref, o_ref):
    o_ref[...] = p_ref[0] + p_ref[1]


_merge = pl.pallas_call(
    _add_body,
    out_shape=jax.ShapeDtypeStruct((N_GRAPHS, FEAT), jnp.float32),
)


@jax.jit
def kernel(x, batch):
    batch3 = batch.reshape(N_CHUNKS, NSUB, SUB)
    partials = _sc_segment_sum(x, batch3)
    return _merge(partials)
